# Initial kernel scaffold; baseline (speedup 1.0000x reference)
import jax
import jax.numpy as jnp
from jax.experimental import pallas as pl


def kernel(x_author, x_paper, edge_index_writes, edge_index_rev_writes, edge_index_cites, W_author, b_author, W_paper, b_paper, a_src_writes, a_dst_writes, a_src_rev, a_dst_rev, a_src_cites, a_dst_cites, k_W, k_b, q):
    raise NotImplementedError("write your pallas kernel here")


# speedup vs baseline: 40.1046x; 40.1046x over previous
import functools

import jax
import jax.numpy as jnp
from jax import lax
from jax.experimental import pallas as pl
from jax.experimental.pallas import tpu as pltpu
from jax.experimental.pallas import tpu_sc as plsc

N_NODES = 10000
D_IN = 128
HID = 128
HEADS = 8
D_HEAD = HID // HEADS
E_EDGES = 160000

NC = 2
NS = 16
NW = NC * NS

CHUNK = 32
NCHUNKS = E_EDGES // CHUNK
ZROWS = 200
NZCHUNKS = N_NODES // ZROWS
ZB = 8
TPACK = N_NODES // 8
TPAD = 1280
TSTAGE = TPAD // NS

_f32 = jnp.float32


_BLK1 = 1000
_G1 = N_NODES // _BLK1


def _proj_body(xa, xp, Wa, ba, Wp, bp, asw, adw, asr, adr, asc, adc,
               ha_o, hp_o, tw, tr, tc):
    ha = jnp.dot(xa[...], Wa[...], preferred_element_type=_f32) + ba[...]
    hp = jnp.dot(xp[...], Wp[...], preferred_element_type=_f32) + bp[...]
    ha_o[...] = ha
    hp_o[...] = hp
    ii = lax.broadcasted_iota(jnp.int32, (HID, 16), 0) // D_HEAD
    jj = lax.broadcasted_iota(jnp.int32, (HID, 16), 1)
    S_lo = (ii == jj).astype(_f32)
    S_hi = (ii + HEADS == jj).astype(_f32)
    for out_ref, hs, a_s, hd, a_d in ((tw, ha, asw, hp, adw),
                                      (tr, hp, asr, ha, adr),
                                      (tc, hp, asc, hp, adc)):
        out_ref[...] = (jnp.dot(hs * a_s[...], S_lo, preferred_element_type=_f32)
                        + jnp.dot(hd * a_d[...], S_hi, preferred_element_type=_f32))


def _run_proj(xa, xp, Wa, ba, Wp, bp, avecs):
    row_spec = pl.BlockSpec((_BLK1, D_IN), lambda i: (i, 0))
    mat_spec = pl.BlockSpec((D_IN, HID), lambda i: (0, 0))
    vec_spec = pl.BlockSpec((1, HID), lambda i: (0, 0))
    tab_spec = pl.BlockSpec((_BLK1, 16), lambda i: (i, 0))
    return pl.pallas_call(
        _proj_body,
        grid=(_G1,),
        in_specs=[row_spec, row_spec, mat_spec, vec_spec, mat_spec, vec_spec]
                 + [vec_spec] * 6,
        out_specs=[row_spec, row_spec] + [tab_spec] * 3,
        out_shape=[jax.ShapeDtypeStruct((N_NODES, HID), _f32)] * 2
                  + [jax.ShapeDtypeStruct((N_NODES, 16), _f32)] * 3,
    )(xa, xp, Wa, ba.reshape(1, HID), Wp, bp.reshape(1, HID),
      *[a.reshape(1, HID) for a in avecs])


def _sc_edge_body(ha, hp, tw, tr, tc,
                  esw, edw, esr, edr, esc, edc,
                  U_out, d_out,
                  U_sh, d_sh, tab_sh,
                  sidx, didx, spb, dpb, scb, dcb,
                  hbuf, ssb, sdb, ebuf, zbuf,
                  sem1, sem2, sem3):
    cid = lax.axis_index("c")
    sid = lax.axis_index("s")
    wid = sid * NC + cid

    zeros16 = jnp.zeros((16,), _f32)

    def zero_zbufs():
        def zrow(i, _):
            for g in range(HID // 16):
                zbuf[i, 16 * g:16 * (g + 1)] = zeros16
            return 0
        lax.fori_loop(0, ZB, zrow, 0)

    zero_zbufs()

    types = ((esw, edw, ha, tw),
             (esr, edr, hp, tr),
             (esc, edc, hp, tc))

    nrows_trip = (NZCHUNKS - sid + NS - 1) // NS
    perm = (lax.iota(jnp.int32, 16) + 8) & 15

    for t, (e_src, e_dst, h_src, st_tab) in enumerate(types):
        def zero_body(r, _):
            rb = (sid + r * NS) * ZROWS
            for z in range(ZROWS // ZB):
                pltpu.sync_copy(zbuf, U_sh.at[pl.ds(rb + z * ZB, ZB)])
            return 0
        lax.fori_loop(0, nrows_trip, zero_body, 0)
        def zero_den(z, _):
            pltpu.sync_copy(zbuf, d_sh.at[pl.ds(sid * TSTAGE + z * ZB, ZB)])
            return 0
        lax.fori_loop(0, TSTAGE // ZB, zero_den, 0)
        pltpu.sync_copy(st_tab.at[pl.ds(sid * TSTAGE, TSTAGE)],
                        tab_sh.at[pl.ds(sid * TSTAGE, TSTAGE)])
        plsc.subcore_barrier()

        ntr = (NCHUNKS - wid + NW - 1) // NW

        def chunk_body(k, _):
            base = (wid + k * NW) * CHUNK
            pltpu.sync_copy(e_src.at[pl.ds(base, CHUNK)], sidx)
            pltpu.sync_copy(e_dst.at[pl.ds(base, CHUNK)], didx)

            def grp(j, _):
                sv = sidx[pl.ds(j * 16, 16)]
                dv = didx[pl.ds(j * 16, 16)]
                spb[pl.ds(j * 16, 16)] = lax.shift_right_logical(sv, 3)
                dpb[pl.ds(j * 16, 16)] = lax.shift_right_logical(dv, 3)
                scb[pl.ds(j * 16, 16)] = lax.shift_left(sv & 7, 4)
                dcb[pl.ds(j * 16, 16)] = lax.shift_left(dv & 7, 4)
                return 0
            lax.fori_loop(0, CHUNK // 16, grp, 0)

            cp1 = pltpu.async_copy(h_src.at[sidx], hbuf, sem1)
            cp2 = pltpu.async_copy(tab_sh.at[spb], ssb, sem2)
            cp3 = pltpu.async_copy(tab_sh.at[dpb], sdb, sem3)
            cp2.wait()
            cp3.wait()
            cp1.wait()

            def grp2(j, _):
                scv = scb[pl.ds(j * 16, 16)]
                dcv = dcb[pl.ds(j * 16, 16)]
                for jj in range(16):
                    i = j * 16 + jj
                    s_vec = ssb[i, pl.ds(scv[jj], 16)]
                    d_vec = sdb[i, pl.ds(dcv[jj], 16)]
                    a = s_vec + d_vec.at[perm].get(mode="promise_in_bounds")
                    a = jnp.where(a >= 0, a, 0.2 * a)
                    e = jnp.exp(a)
                    for g in range(HID // 16):
                        ebuf[i, 16 * g:16 * (g + 1)] = zeros16
                    ebuf[i, pl.ds(dcv[jj], 16)] = e
                    for g in range(HID // 16):
                        hbuf[i, 16 * g:16 * (g + 1)] = (
                            hbuf[i, 16 * g:16 * (g + 1)] * e[g])
                return 0
            lax.fori_loop(0, CHUNK // 16, grp2, 0)

            pltpu.sync_copy(hbuf, U_sh.at[didx], add=True)
            pltpu.sync_copy(ebuf, d_sh.at[dpb], add=True)
            return 0

        lax.fori_loop(0, ntr, chunk_body, 0)
        plsc.subcore_barrier()

        def wb_body(r, _):
            rb = (sid + r * NS) * ZROWS
            pltpu.sync_copy(U_sh.at[pl.ds(rb, ZROWS)], U_out.at[t, cid, pl.ds(rb, ZROWS)])
            return 0
        lax.fori_loop(0, nrows_trip, wb_body, 0)
        pltpu.sync_copy(d_sh.at[pl.ds(sid * TSTAGE, TSTAGE)],
                        d_out.at[t, cid, pl.ds(sid * TSTAGE, TSTAGE)])


def _run_sc_edges(ha, hp, tables, edges):
    mesh = plsc.VectorSubcoreMesh(core_axis_name="c", subcore_axis_name="s",
                                  num_cores=NC, num_subcores=NS)
    kern = pl.kernel(
        _sc_edge_body,
        out_type=(jax.ShapeDtypeStruct((3, NC, N_NODES, HID), _f32),
                  jax.ShapeDtypeStruct((3, NC, TPAD, HID), _f32)),
        mesh=mesh,
        scratch_types=(
            pltpu.VMEM_SHARED((N_NODES, HID), _f32),
            pltpu.VMEM_SHARED((TPAD, HID), _f32),
            pltpu.VMEM_SHARED((TPAD, HID), _f32),
            pltpu.VMEM((CHUNK,), jnp.int32),
            pltpu.VMEM((CHUNK,), jnp.int32),
            pltpu.VMEM((CHUNK,), jnp.int32),
            pltpu.VMEM((CHUNK,), jnp.int32),
            pltpu.VMEM((CHUNK,), jnp.int32),
            pltpu.VMEM((CHUNK,), jnp.int32),
            pltpu.VMEM((CHUNK, HID), _f32),
            pltpu.VMEM((CHUNK, HID), _f32),
            pltpu.VMEM((CHUNK, HID), _f32),
            pltpu.VMEM((CHUNK, HID), _f32),
            pltpu.VMEM((ZB, HID), _f32),
            pltpu.SemaphoreType.DMA,
            pltpu.SemaphoreType.DMA,
            pltpu.SemaphoreType.DMA,
        ),
    )
    return kern(ha, hp, *tables, *edges)


_BLK2 = 1000
_G2 = N_NODES // _BLK2


def _combine_body(U, den, kW, kb, oa, ow, oc, psw, psc):
    kk = lax.broadcasted_iota(jnp.int32, (16, HID), 0)
    jj = lax.broadcasted_iota(jnp.int32, (16, HID), 1) // D_HEAD
    St = (kk == jj).astype(_f32)
    outs = []
    for t in range(3):
        Ut = U[t, 0] + U[t, 1]
        dent = den[t, 0] + den[t, 1]
        r = 1.0 / (dent + 1e-16)
        rexp = jnp.dot(r, St, preferred_element_type=_f32)
        outs.append(jnp.maximum(Ut * rexp, 0.0))
    ow[...] = outs[0]
    oa[...] = outs[1]
    oc[...] = outs[2]
    gw = jnp.tanh(jnp.dot(outs[0], kW[...], preferred_element_type=_f32) + kb[...])
    gc = jnp.tanh(jnp.dot(outs[2], kW[...], preferred_element_type=_f32) + kb[...])
    psw[...] = jnp.sum(gw, axis=0, keepdims=True)[None]
    psc[...] = jnp.sum(gc, axis=0, keepdims=True)[None]


def _run_combine(U, den, kW, kb):
    U_spec = pl.BlockSpec((3, NC, _BLK2, HID), lambda i: (0, 0, i, 0))
    d_spec = pl.BlockSpec((3, NC, _BLK2, 16), lambda i: (0, 0, i, 0))
    mat_spec = pl.BlockSpec((HID, HID), lambda i: (0, 0))
    vec_spec = pl.BlockSpec((1, HID), lambda i: (0, 0))
    row_spec = pl.BlockSpec((_BLK2, HID), lambda i: (i, 0))
    ps_spec = pl.BlockSpec((1, 1, HID), lambda i: (i, 0, 0))
    return pl.pallas_call(
        _combine_body,
        grid=(_G2,),
        in_specs=[U_spec, d_spec, mat_spec, vec_spec],
        out_specs=[row_spec, row_spec, row_spec, ps_spec, ps_spec],
        out_shape=[jax.ShapeDtypeStruct((N_NODES, HID), _f32)] * 3
                  + [jax.ShapeDtypeStruct((_G2, 1, HID), _f32)] * 2,
    )(U, den, kW, kb.reshape(1, HID))


def _final_body(ow, oc, psw, psc, q, op):
    qv = q[...]
    sw = jnp.sum(psw[...] * qv) / N_NODES
    sc = jnp.sum(psc[...] * qv) / N_NODES
    m = jnp.maximum(sw, sc)
    ew = jnp.exp(sw - m)
    ec = jnp.exp(sc - m)
    inv = 1.0 / (ew + ec)
    op[...] = (ew * inv) * ow[...] + (ec * inv) * oc[...]


def _run_final(ow, oc, psw, psc, q):
    row_spec = pl.BlockSpec((_BLK2, HID), lambda i: (i, 0))
    ps_spec = pl.BlockSpec((_G2, 1, HID), lambda i: (0, 0, 0))
    vec_spec = pl.BlockSpec((1, HID), lambda i: (0, 0))
    return pl.pallas_call(
        _final_body,
        grid=(_G2,),
        in_specs=[row_spec, row_spec, ps_spec, ps_spec, vec_spec],
        out_specs=row_spec,
        out_shape=jax.ShapeDtypeStruct((N_NODES, HID), _f32),
    )(ow, oc, psw, psc, q.reshape(1, HID))


def kernel(x_author, x_paper, edge_index_writes, edge_index_rev_writes,
           edge_index_cites, W_author, b_author, W_paper, b_paper,
           a_src_writes, a_dst_writes, a_src_rev, a_dst_rev, a_src_cites,
           a_dst_cites, k_W, k_b, q):
    avecs = (a_src_writes, a_dst_writes, a_src_rev, a_dst_rev,
             a_src_cites, a_dst_cites)
    ha, hp, tw, tr, tc = _run_proj(
        x_author, x_paper, W_author, b_author, W_paper, b_paper, avecs)

    edges = (edge_index_writes[0], edge_index_writes[1],
             edge_index_rev_writes[0], edge_index_rev_writes[1],
             edge_index_cites[0], edge_index_cites[1])
    pack = lambda tab: jnp.pad(tab.reshape(TPACK, 8 * 16), ((0, TPAD - TPACK), (0, 0)))
    U, den_packed = _run_sc_edges(ha, hp, (pack(tw), pack(tr), pack(tc)), edges)
    den = den_packed[:, :, :TPACK].reshape(3, NC, N_NODES, 16)

    oa, ow, oc, psw, psc = _run_combine(U, den, k_W, k_b)
    op = _run_final(ow, oc, psw, psc, q)
    return (oa, op)


# speedup vs baseline: 44.7288x; 1.1153x over previous
import functools

import jax
import jax.numpy as jnp
from jax import lax
from jax.experimental import pallas as pl
from jax.experimental.pallas import tpu as pltpu
from jax.experimental.pallas import tpu_sc as plsc

N_NODES = 10000
D_IN = 128
HID = 128
HEADS = 8
D_HEAD = HID // HEADS
E_EDGES = 160000

NC = 2
NS = 16
NW = NC * NS

CHUNK = 32
NCHUNKS = E_EDGES // CHUNK
ZROWS = 200
NZCHUNKS = N_NODES // ZROWS
ZB = 8
TPACK = N_NODES // 8
TPAD = 1280
TSTAGE = TPAD // NS

_f32 = jnp.float32


_BLK1 = 1000
_G1 = N_NODES // _BLK1


def _proj_body(xa, xp, Wa, ba, Wp, bp, asw, adw, asr, adr, asc, adc,
               ha_o, hp_o, tw, tr, tc):
    ha = jnp.dot(xa[...], Wa[...], preferred_element_type=_f32) + ba[...]
    hp = jnp.dot(xp[...], Wp[...], preferred_element_type=_f32) + bp[...]
    ha_o[...] = ha
    hp_o[...] = hp
    ii = lax.broadcasted_iota(jnp.int32, (HID, 16), 0) // D_HEAD
    jj = lax.broadcasted_iota(jnp.int32, (HID, 16), 1)
    S_lo = (ii == jj).astype(_f32)
    S_hi = (ii + HEADS == jj).astype(_f32)
    for out_ref, hs, a_s, hd, a_d in ((tw, ha, asw, hp, adw),
                                      (tr, hp, asr, ha, adr),
                                      (tc, hp, asc, hp, adc)):
        out_ref[...] = (jnp.dot(hs * a_s[...], S_lo, preferred_element_type=_f32)
                        + jnp.dot(hd * a_d[...], S_hi, preferred_element_type=_f32))


def _run_proj(xa, xp, Wa, ba, Wp, bp, avecs):
    row_spec = pl.BlockSpec((_BLK1, D_IN), lambda i: (i, 0))
    mat_spec = pl.BlockSpec((D_IN, HID), lambda i: (0, 0))
    vec_spec = pl.BlockSpec((1, HID), lambda i: (0, 0))
    tab_spec = pl.BlockSpec((_BLK1, 16), lambda i: (i, 0))
    return pl.pallas_call(
        _proj_body,
        grid=(_G1,),
        in_specs=[row_spec, row_spec, mat_spec, vec_spec, mat_spec, vec_spec]
                 + [vec_spec] * 6,
        out_specs=[row_spec, row_spec] + [tab_spec] * 3,
        out_shape=[jax.ShapeDtypeStruct((N_NODES, HID), _f32)] * 2
                  + [jax.ShapeDtypeStruct((N_NODES, 16), _f32)] * 3,
    )(xa, xp, Wa, ba.reshape(1, HID), Wp, bp.reshape(1, HID),
      *[a.reshape(1, HID) for a in avecs])


def _sc_edge_body(ha, hp, tw, tr, tc,
                  esw, edw, esr, edr, esc, edc,
                  U_out, d_out,
                  U_sh, d_sh, tab_sh,
                  sidxA, didxA, spbA, dpbA, scbA, dcbA, hbufA, ebufA,
                  sidxB, didxB, spbB, dpbB, scbB, dcbB, hbufB, ebufB,
                  ssb, sdb, zbuf,
                  semA1, semB1, sem2, sem3):
    cid = lax.axis_index("c")
    sid = lax.axis_index("s")
    wid = sid * NC + cid

    zeros16 = jnp.zeros((16,), _f32)

    def zero_zbufs():
        def zrow(i, _):
            for g in range(HID // 16):
                zbuf[i, 16 * g:16 * (g + 1)] = zeros16
            return 0
        lax.fori_loop(0, ZB, zrow, 0)

    zero_zbufs()

    types = ((esw, edw, ha, tw),
             (esr, edr, hp, tr),
             (esc, edc, hp, tc))

    nrows_trip = (NZCHUNKS - sid + NS - 1) // NS
    perm = (lax.iota(jnp.int32, 16) + 8) & 15

    for t, (e_src, e_dst, h_src, st_tab) in enumerate(types):
        def zero_body(r, _):
            rb = (sid + r * NS) * ZROWS
            for z in range(ZROWS // ZB):
                pltpu.sync_copy(zbuf, U_sh.at[pl.ds(rb + z * ZB, ZB)])
            return 0
        lax.fori_loop(0, nrows_trip, zero_body, 0)
        def zero_den(z, _):
            pltpu.sync_copy(zbuf, d_sh.at[pl.ds(sid * TSTAGE + z * ZB, ZB)])
            return 0
        lax.fori_loop(0, TSTAGE // ZB, zero_den, 0)
        pltpu.sync_copy(st_tab.at[pl.ds(sid * TSTAGE, TSTAGE)],
                        tab_sh.at[pl.ds(sid * TSTAGE, TSTAGE)])
        plsc.subcore_barrier()

        ntr = (NCHUNKS - wid + NW - 1) // NW

        bufsets = (
            (sidxA, didxA, spbA, dpbA, scbA, dcbA, hbufA, ebufA, semA1),
            (sidxB, didxB, spbB, dpbB, scbB, dcbB, hbufB, ebufB, semB1),
        )

        def prefetch(P, k):
            (sidx, didx, spb, dpb, scb, dcb, hbuf, ebuf, s1) = P
            base = (wid + k * NW) * CHUNK
            pltpu.sync_copy(e_src.at[pl.ds(base, CHUNK)], sidx)
            pltpu.sync_copy(e_dst.at[pl.ds(base, CHUNK)], didx)

            def grp(j, _):
                sv = sidx[pl.ds(j * 16, 16)]
                dv = didx[pl.ds(j * 16, 16)]
                spb[pl.ds(j * 16, 16)] = lax.shift_right_logical(sv, 3)
                dpb[pl.ds(j * 16, 16)] = lax.shift_right_logical(dv, 3)
                scb[pl.ds(j * 16, 16)] = lax.shift_left(sv & 7, 4)
                dcb[pl.ds(j * 16, 16)] = lax.shift_left(dv & 7, 4)
                return 0
            lax.fori_loop(0, CHUNK // 16, grp, 0)

            pltpu.async_copy(h_src.at[sidx], hbuf, s1)

        def process(P):
            (sidx, didx, spb, dpb, scb, dcb, hbuf, ebuf, s1) = P
            cp2 = pltpu.async_copy(tab_sh.at[spb], ssb, sem2)
            cp3 = pltpu.async_copy(tab_sh.at[dpb], sdb, sem3)
            cp2.wait()
            cp3.wait()
            pltpu.make_async_copy(h_src.at[sidx], hbuf, s1).wait()

            def grp2(j, _):
                scv = scb[pl.ds(j * 16, 16)]
                dcv = dcb[pl.ds(j * 16, 16)]
                for jj in range(16):
                    i = j * 16 + jj
                    s_vec = ssb[i, pl.ds(scv[jj], 16)]
                    d_vec = sdb[i, pl.ds(dcv[jj], 16)]
                    a = s_vec + d_vec.at[perm].get(mode="promise_in_bounds")
                    a = jnp.where(a >= 0, a, 0.2 * a)
                    e = jnp.exp(a)
                    for g in range(HID // 16):
                        ebuf[i, 16 * g:16 * (g + 1)] = zeros16
                    ebuf[i, pl.ds(dcv[jj], 16)] = e
                    for g in range(HID // 16):
                        hbuf[i, 16 * g:16 * (g + 1)] = (
                            hbuf[i, 16 * g:16 * (g + 1)] * e[g])
                return 0
            lax.fori_loop(0, CHUNK // 16, grp2, 0)

            pltpu.sync_copy(hbuf, U_sh.at[didx], add=True)
            pltpu.sync_copy(ebuf, d_sh.at[dpb], add=True)

        prefetch(bufsets[0], 0)
        npairs = (ntr + 1) // 2

        def pair_body(m, _):
            k1 = 2 * m + 1

            @pl.when(k1 < ntr)
            def _():
                prefetch(bufsets[1], k1)
            process(bufsets[0])

            @pl.when(k1 + 1 < ntr)
            def _():
                prefetch(bufsets[0], k1 + 1)

            @pl.when(k1 < ntr)
            def _():
                process(bufsets[1])
            return 0

        lax.fori_loop(0, npairs, pair_body, 0)
        plsc.subcore_barrier()

        def wb_body(r, _):
            rb = (sid + r * NS) * ZROWS
            pltpu.sync_copy(U_sh.at[pl.ds(rb, ZROWS)], U_out.at[t, cid, pl.ds(rb, ZROWS)])
            return 0
        lax.fori_loop(0, nrows_trip, wb_body, 0)
        pltpu.sync_copy(d_sh.at[pl.ds(sid * TSTAGE, TSTAGE)],
                        d_out.at[t, cid, pl.ds(sid * TSTAGE, TSTAGE)])


def _run_sc_edges(ha, hp, tables, edges):
    mesh = plsc.VectorSubcoreMesh(core_axis_name="c", subcore_axis_name="s",
                                  num_cores=NC, num_subcores=NS)
    kern = pl.kernel(
        _sc_edge_body,
        out_type=(jax.ShapeDtypeStruct((3, NC, N_NODES, HID), _f32),
                  jax.ShapeDtypeStruct((3, NC, TPAD, HID), _f32)),
        mesh=mesh,
        scratch_types=(
            pltpu.VMEM_SHARED((N_NODES, HID), _f32),
            pltpu.VMEM_SHARED((TPAD, HID), _f32),
            pltpu.VMEM_SHARED((TPAD, HID), _f32),
        ) + tuple(
            [pltpu.VMEM((CHUNK,), jnp.int32)] * 6
            + [pltpu.VMEM((CHUNK, HID), _f32)] * 2
            + [pltpu.VMEM((CHUNK,), jnp.int32)] * 6
            + [pltpu.VMEM((CHUNK, HID), _f32)] * 2
            + [pltpu.VMEM((CHUNK, HID), _f32)] * 2
        ) + (
            pltpu.VMEM((ZB, HID), _f32),
            pltpu.SemaphoreType.DMA,
            pltpu.SemaphoreType.DMA,
            pltpu.SemaphoreType.DMA,
            pltpu.SemaphoreType.DMA,
        ),
    )
    return kern(ha, hp, *tables, *edges)


_BLK2 = 1000
_G2 = N_NODES // _BLK2


def _combine_body(U, den, kW, kb, oa, ow, oc, psw, psc):
    kk = lax.broadcasted_iota(jnp.int32, (16, HID), 0)
    jj = lax.broadcasted_iota(jnp.int32, (16, HID), 1) // D_HEAD
    St = (kk == jj).astype(_f32)
    outs = []
    for t in range(3):
        Ut = U[t, 0] + U[t, 1]
        dent = den[t, 0] + den[t, 1]
        r = 1.0 / (dent + 1e-16)
        rexp = jnp.dot(r, St, preferred_element_type=_f32)
        outs.append(jnp.maximum(Ut * rexp, 0.0))
    ow[...] = outs[0]
    oa[...] = outs[1]
    oc[...] = outs[2]
    gw = jnp.tanh(jnp.dot(outs[0], kW[...], preferred_element_type=_f32) + kb[...])
    gc = jnp.tanh(jnp.dot(outs[2], kW[...], preferred_element_type=_f32) + kb[...])
    psw[...] = jnp.sum(gw, axis=0, keepdims=True)[None]
    psc[...] = jnp.sum(gc, axis=0, keepdims=True)[None]


def _run_combine(U, den, kW, kb):
    U_spec = pl.BlockSpec((3, NC, _BLK2, HID), lambda i: (0, 0, i, 0))
    d_spec = pl.BlockSpec((3, NC, _BLK2, 16), lambda i: (0, 0, i, 0))
    mat_spec = pl.BlockSpec((HID, HID), lambda i: (0, 0))
    vec_spec = pl.BlockSpec((1, HID), lambda i: (0, 0))
    row_spec = pl.BlockSpec((_BLK2, HID), lambda i: (i, 0))
    ps_spec = pl.BlockSpec((1, 1, HID), lambda i: (i, 0, 0))
    return pl.pallas_call(
        _combine_body,
        grid=(_G2,),
        in_specs=[U_spec, d_spec, mat_spec, vec_spec],
        out_specs=[row_spec, row_spec, row_spec, ps_spec, ps_spec],
        out_shape=[jax.ShapeDtypeStruct((N_NODES, HID), _f32)] * 3
                  + [jax.ShapeDtypeStruct((_G2, 1, HID), _f32)] * 2,
    )(U, den, kW, kb.reshape(1, HID))


def _final_body(ow, oc, psw, psc, q, op):
    qv = q[...]
    sw = jnp.sum(psw[...] * qv) / N_NODES
    sc = jnp.sum(psc[...] * qv) / N_NODES
    m = jnp.maximum(sw, sc)
    ew = jnp.exp(sw - m)
    ec = jnp.exp(sc - m)
    inv = 1.0 / (ew + ec)
    op[...] = (ew * inv) * ow[...] + (ec * inv) * oc[...]


def _run_final(ow, oc, psw, psc, q):
    row_spec = pl.BlockSpec((_BLK2, HID), lambda i: (i, 0))
    ps_spec = pl.BlockSpec((_G2, 1, HID), lambda i: (0, 0, 0))
    vec_spec = pl.BlockSpec((1, HID), lambda i: (0, 0))
    return pl.pallas_call(
        _final_body,
        grid=(_G2,),
        in_specs=[row_spec, row_spec, ps_spec, ps_spec, vec_spec],
        out_specs=row_spec,
        out_shape=jax.ShapeDtypeStruct((N_NODES, HID), _f32),
    )(ow, oc, psw, psc, q.reshape(1, HID))


def kernel(x_author, x_paper, edge_index_writes, edge_index_rev_writes,
           edge_index_cites, W_author, b_author, W_paper, b_paper,
           a_src_writes, a_dst_writes, a_src_rev, a_dst_rev, a_src_cites,
           a_dst_cites, k_W, k_b, q):
    avecs = (a_src_writes, a_dst_writes, a_src_rev, a_dst_rev,
             a_src_cites, a_dst_cites)
    ha, hp, tw, tr, tc = _run_proj(
        x_author, x_paper, W_author, b_author, W_paper, b_paper, avecs)

    edges = (edge_index_writes[0], edge_index_writes[1],
             edge_index_rev_writes[0], edge_index_rev_writes[1],
             edge_index_cites[0], edge_index_cites[1])
    pack = lambda tab: jnp.pad(tab.reshape(TPACK, 8 * 16), ((0, TPAD - TPACK), (0, 0)))
    U, den_packed = _run_sc_edges(ha, hp, (pack(tw), pack(tr), pack(tc)), edges)
    den = den_packed[:, :, :TPACK].reshape(3, NC, N_NODES, 16)

    oa, ow, oc, psw, psc = _run_combine(U, den, k_W, k_b)
    op = _run_final(ow, oc, psw, psc, q)
    return (oa, op)


# speedup vs baseline: 57.1352x; 1.2774x over previous
import functools

import jax
import jax.numpy as jnp
from jax import lax
from jax.experimental import pallas as pl
from jax.experimental.pallas import tpu as pltpu
from jax.experimental.pallas import tpu_sc as plsc

N_NODES = 10000
D_IN = 128
HID = 128
HEADS = 8
D_HEAD = HID // HEADS
E_EDGES = 160000

NC = 2
NS = 16
NW = NC * NS

CHUNK = 32
NCHUNKS = E_EDGES // CHUNK
ZROWS = 200
NZCHUNKS = N_NODES // ZROWS
ZB = 8
TPACK = N_NODES // 8
TPAD = 1280
TSTAGE = TPAD // NS
EBCH = 16
EBLK = EBCH * CHUNK
EPAD = 157 * CHUNK * NW - E_EDGES + EBLK

_f32 = jnp.float32


_BLK1 = 1000
_G1 = N_NODES // _BLK1


def _proj_body(xa, xp, Wa, ba, Wp, bp, asw, adw, asr, adr, asc, adc,
               ha_o, hp_o, tw, tr, tc):
    ha = jnp.dot(xa[...], Wa[...], preferred_element_type=_f32) + ba[...]
    hp = jnp.dot(xp[...], Wp[...], preferred_element_type=_f32) + bp[...]
    ha_o[...] = ha
    hp_o[...] = hp
    ii = lax.broadcasted_iota(jnp.int32, (HID, 16), 0) // D_HEAD
    jj = lax.broadcasted_iota(jnp.int32, (HID, 16), 1)
    S_lo = (ii == jj).astype(_f32)
    S_hi = (ii + HEADS == jj).astype(_f32)
    for out_ref, hs, a_s, hd, a_d in ((tw, ha, asw, hp, adw),
                                      (tr, hp, asr, ha, adr),
                                      (tc, hp, asc, hp, adc)):
        out_ref[...] = (jnp.dot(hs * a_s[...], S_lo, preferred_element_type=_f32)
                        + jnp.dot(hd * a_d[...], S_hi, preferred_element_type=_f32))


def _run_proj(xa, xp, Wa, ba, Wp, bp, avecs):
    row_spec = pl.BlockSpec((_BLK1, D_IN), lambda i: (i, 0))
    mat_spec = pl.BlockSpec((D_IN, HID), lambda i: (0, 0))
    vec_spec = pl.BlockSpec((1, HID), lambda i: (0, 0))
    tab_spec = pl.BlockSpec((_BLK1, 16), lambda i: (i, 0))
    return pl.pallas_call(
        _proj_body,
        grid=(_G1,),
        in_specs=[row_spec, row_spec, mat_spec, vec_spec, mat_spec, vec_spec]
                 + [vec_spec] * 6,
        out_specs=[row_spec, row_spec] + [tab_spec] * 3,
        out_shape=[jax.ShapeDtypeStruct((N_NODES, HID), _f32)] * 2
                  + [jax.ShapeDtypeStruct((N_NODES, 16), _f32)] * 3,
    )(xa, xp, Wa, ba.reshape(1, HID), Wp, bp.reshape(1, HID),
      *[a.reshape(1, HID) for a in avecs])


def _sc_edge_body(ha, hp, tw, tr, tc,
                  esw, edw, esr, edr, esc, edc,
                  U_out, d_out,
                  U_sh, d_sh, tab_sh,
                  sidxA, didxA, spbA, dpbA, scbA, dcbA, hbufA, ebufA,
                  sidxB, didxB, spbB, dpbB, scbB, dcbB, hbufB, ebufB,
                  ssb, sdb, zbuf, sidx_all, didx_all,
                  semA1, semB1, sem2, sem3):
    cid = lax.axis_index("c")
    sid = lax.axis_index("s")
    wid = sid * NC + cid

    zeros16 = jnp.zeros((16,), _f32)

    def zero_zbufs():
        def zrow(i, _):
            for g in range(HID // 16):
                zbuf[i, 16 * g:16 * (g + 1)] = zeros16
            return 0
        lax.fori_loop(0, ZB, zrow, 0)

    zero_zbufs()

    types = ((esw, edw, ha, tw),
             (esr, edr, hp, tr),
             (esc, edc, hp, tc))

    nrows_trip = (NZCHUNKS - sid + NS - 1) // NS
    perm = (lax.iota(jnp.int32, 16) + 8) & 15

    for t, (e_src, e_dst, h_src, st_tab) in enumerate(types):
        def zero_body(r, _):
            rb = (sid + r * NS) * ZROWS
            for z in range(ZROWS // ZB):
                pltpu.sync_copy(zbuf, U_sh.at[pl.ds(rb + z * ZB, ZB)])
            return 0
        lax.fori_loop(0, nrows_trip, zero_body, 0)
        def zero_den(z, _):
            pltpu.sync_copy(zbuf, d_sh.at[pl.ds(sid * TSTAGE + z * ZB, ZB)])
            return 0
        lax.fori_loop(0, TSTAGE // ZB, zero_den, 0)
        pltpu.sync_copy(st_tab.at[pl.ds(sid * TSTAGE, TSTAGE)],
                        tab_sh.at[pl.ds(sid * TSTAGE, TSTAGE)])
        plsc.subcore_barrier()

        ntr = 156 + (wid < 8).astype(jnp.int32)
        start_w = wid * 156 + jnp.minimum(wid, 8)

        bufsets = (
            (sidxA, didxA, spbA, dpbA, scbA, dcbA, hbufA, ebufA, semA1),
            (sidxB, didxB, spbB, dpbB, scbB, dcbB, hbufB, ebufB, semB1),
        )

        def prefetch(P, k):
            (sidx, didx, spb, dpb, scb, dcb, hbuf, ebuf, s1) = P

            @pl.when(k % EBCH == 0)
            def _():
                eb = start_w * CHUNK + (k // EBCH) * EBLK
                pltpu.sync_copy(e_src.at[pl.ds(eb, EBLK)], sidx_all)
                pltpu.sync_copy(e_dst.at[pl.ds(eb, EBLK)], didx_all)
            off = (k % EBCH) * CHUNK

            def grp(j, _):
                sv = sidx_all[pl.ds(off + j * 16, 16)]
                dv = didx_all[pl.ds(off + j * 16, 16)]
                sidx[pl.ds(j * 16, 16)] = sv
                didx[pl.ds(j * 16, 16)] = dv
                spb[pl.ds(j * 16, 16)] = lax.shift_right_logical(sv, 3)
                dpb[pl.ds(j * 16, 16)] = lax.shift_right_logical(dv, 3)
                scb[pl.ds(j * 16, 16)] = lax.shift_left(sv & 7, 4)
                dcb[pl.ds(j * 16, 16)] = lax.shift_left(dv & 7, 4)
                return 0
            lax.fori_loop(0, CHUNK // 16, grp, 0)

            pltpu.async_copy(h_src.at[sidx], hbuf, s1)

        def process(P):
            (sidx, didx, spb, dpb, scb, dcb, hbuf, ebuf, s1) = P
            cp2 = pltpu.async_copy(tab_sh.at[spb], ssb, sem2)
            cp3 = pltpu.async_copy(tab_sh.at[dpb], sdb, sem3)
            cp2.wait()
            cp3.wait()
            pltpu.make_async_copy(h_src.at[sidx], hbuf, s1).wait()

            def grp2(j, _):
                scv = scb[pl.ds(j * 16, 16)]
                dcv = dcb[pl.ds(j * 16, 16)]
                for jj in range(16):
                    i = j * 16 + jj
                    s_vec = ssb[i, pl.ds(scv[jj], 16)]
                    d_vec = sdb[i, pl.ds(dcv[jj], 16)]
                    a = s_vec + d_vec.at[perm].get(mode="promise_in_bounds")
                    a = jnp.where(a >= 0, a, 0.2 * a)
                    e = jnp.exp(a)
                    for g in range(HID // 16):
                        ebuf[i, 16 * g:16 * (g + 1)] = zeros16
                    ebuf[i, pl.ds(dcv[jj], 16)] = e
                    for g in range(HID // 16):
                        hbuf[i, 16 * g:16 * (g + 1)] = (
                            hbuf[i, 16 * g:16 * (g + 1)] * e[g])
                return 0
            lax.fori_loop(0, CHUNK // 16, grp2, 0)

            pltpu.sync_copy(hbuf, U_sh.at[didx], add=True)
            pltpu.sync_copy(ebuf, d_sh.at[dpb], add=True)

        prefetch(bufsets[0], 0)
        npairs = (ntr + 1) // 2

        def pair_body(m, _):
            k1 = 2 * m + 1

            @pl.when(k1 < ntr)
            def _():
                prefetch(bufsets[1], k1)
            process(bufsets[0])

            @pl.when(k1 + 1 < ntr)
            def _():
                prefetch(bufsets[0], k1 + 1)

            @pl.when(k1 < ntr)
            def _():
                process(bufsets[1])
            return 0

        lax.fori_loop(0, npairs, pair_body, 0)
        plsc.subcore_barrier()

        def wb_body(r, _):
            rb = (sid + r * NS) * ZROWS
            pltpu.sync_copy(U_sh.at[pl.ds(rb, ZROWS)], U_out.at[t, cid, pl.ds(rb, ZROWS)])
            return 0
        lax.fori_loop(0, nrows_trip, wb_body, 0)
        pltpu.sync_copy(d_sh.at[pl.ds(sid * TSTAGE, TSTAGE)],
                        d_out.at[t, cid, pl.ds(sid * TSTAGE, TSTAGE)])


def _run_sc_edges(ha, hp, tables, edges):
    mesh = plsc.VectorSubcoreMesh(core_axis_name="c", subcore_axis_name="s",
                                  num_cores=NC, num_subcores=NS)
    kern = pl.kernel(
        _sc_edge_body,
        out_type=(jax.ShapeDtypeStruct((3, NC, N_NODES, HID), _f32),
                  jax.ShapeDtypeStruct((3, NC, TPAD, HID), _f32)),
        mesh=mesh,
        scratch_types=(
            pltpu.VMEM_SHARED((N_NODES, HID), _f32),
            pltpu.VMEM_SHARED((TPAD, HID), _f32),
            pltpu.VMEM_SHARED((TPAD, HID), _f32),
        ) + tuple(
            [pltpu.VMEM((CHUNK,), jnp.int32)] * 6
            + [pltpu.VMEM((CHUNK, HID), _f32)] * 2
            + [pltpu.VMEM((CHUNK,), jnp.int32)] * 6
            + [pltpu.VMEM((CHUNK, HID), _f32)] * 2
            + [pltpu.VMEM((CHUNK, HID), _f32)] * 2
        ) + (
            pltpu.VMEM((ZB, HID), _f32),
            pltpu.VMEM((EBLK,), jnp.int32),
            pltpu.VMEM((EBLK,), jnp.int32),
            pltpu.SemaphoreType.DMA,
            pltpu.SemaphoreType.DMA,
            pltpu.SemaphoreType.DMA,
            pltpu.SemaphoreType.DMA,
        ),
    )
    return kern(ha, hp, *tables, *edges)


_BLK2 = 1000
_G2 = N_NODES // _BLK2


def _combine_body(U, den, kW, kb, oa, ow, oc, psw, psc):
    kk = lax.broadcasted_iota(jnp.int32, (16, HID), 0)
    jj = lax.broadcasted_iota(jnp.int32, (16, HID), 1) // D_HEAD
    St = (kk == jj).astype(_f32)
    outs = []
    for t in range(3):
        Ut = U[t, 0] + U[t, 1]
        dent = den[t, 0] + den[t, 1]
        r = 1.0 / (dent + 1e-16)
        rexp = jnp.dot(r, St, preferred_element_type=_f32)
        outs.append(jnp.maximum(Ut * rexp, 0.0))
    ow[...] = outs[0]
    oa[...] = outs[1]
    oc[...] = outs[2]
    gw = jnp.tanh(jnp.dot(outs[0], kW[...], preferred_element_type=_f32) + kb[...])
    gc = jnp.tanh(jnp.dot(outs[2], kW[...], preferred_element_type=_f32) + kb[...])
    psw[...] = jnp.sum(gw, axis=0, keepdims=True)[None]
    psc[...] = jnp.sum(gc, axis=0, keepdims=True)[None]


def _run_combine(U, den, kW, kb):
    U_spec = pl.BlockSpec((3, NC, _BLK2, HID), lambda i: (0, 0, i, 0))
    d_spec = pl.BlockSpec((3, NC, _BLK2, 16), lambda i: (0, 0, i, 0))
    mat_spec = pl.BlockSpec((HID, HID), lambda i: (0, 0))
    vec_spec = pl.BlockSpec((1, HID), lambda i: (0, 0))
    row_spec = pl.BlockSpec((_BLK2, HID), lambda i: (i, 0))
    ps_spec = pl.BlockSpec((1, 1, HID), lambda i: (i, 0, 0))
    return pl.pallas_call(
        _combine_body,
        grid=(_G2,),
        in_specs=[U_spec, d_spec, mat_spec, vec_spec],
        out_specs=[row_spec, row_spec, row_spec, ps_spec, ps_spec],
        out_shape=[jax.ShapeDtypeStruct((N_NODES, HID), _f32)] * 3
                  + [jax.ShapeDtypeStruct((_G2, 1, HID), _f32)] * 2,
    )(U, den, kW, kb.reshape(1, HID))


def _final_body(ow, oc, psw, psc, q, op):
    qv = q[...]
    sw = jnp.sum(psw[...] * qv) / N_NODES
    sc = jnp.sum(psc[...] * qv) / N_NODES
    m = jnp.maximum(sw, sc)
    ew = jnp.exp(sw - m)
    ec = jnp.exp(sc - m)
    inv = 1.0 / (ew + ec)
    op[...] = (ew * inv) * ow[...] + (ec * inv) * oc[...]


def _run_final(ow, oc, psw, psc, q):
    row_spec = pl.BlockSpec((_BLK2, HID), lambda i: (i, 0))
    ps_spec = pl.BlockSpec((_G2, 1, HID), lambda i: (0, 0, 0))
    vec_spec = pl.BlockSpec((1, HID), lambda i: (0, 0))
    return pl.pallas_call(
        _final_body,
        grid=(_G2,),
        in_specs=[row_spec, row_spec, ps_spec, ps_spec, vec_spec],
        out_specs=row_spec,
        out_shape=jax.ShapeDtypeStruct((N_NODES, HID), _f32),
    )(ow, oc, psw, psc, q.reshape(1, HID))


def kernel(x_author, x_paper, edge_index_writes, edge_index_rev_writes,
           edge_index_cites, W_author, b_author, W_paper, b_paper,
           a_src_writes, a_dst_writes, a_src_rev, a_dst_rev, a_src_cites,
           a_dst_cites, k_W, k_b, q):
    avecs = (a_src_writes, a_dst_writes, a_src_rev, a_dst_rev,
             a_src_cites, a_dst_cites)
    ha, hp, tw, tr, tc = _run_proj(
        x_author, x_paper, W_author, b_author, W_paper, b_paper, avecs)

    epad = lambda e: jnp.pad(e, (0, EPAD))
    edges = (epad(edge_index_writes[0]), epad(edge_index_writes[1]),
             epad(edge_index_rev_writes[0]), epad(edge_index_rev_writes[1]),
             epad(edge_index_cites[0]), epad(edge_index_cites[1]))
    pack = lambda tab: jnp.pad(tab.reshape(TPACK, 8 * 16), ((0, TPAD - TPACK), (0, 0)))
    U, den_packed = _run_sc_edges(ha, hp, (pack(tw), pack(tr), pack(tc)), edges)
    den = den_packed[:, :, :TPACK].reshape(3, NC, N_NODES, 16)

    oa, ow, oc, psw, psc = _run_combine(U, den, k_W, k_b)
    op = _run_final(ow, oc, psw, psc, q)
    return (oa, op)


# speedup vs baseline: 58.6202x; 1.0260x over previous
import functools

import jax
import jax.numpy as jnp
from jax import lax
from jax.experimental import pallas as pl
from jax.experimental.pallas import tpu as pltpu
from jax.experimental.pallas import tpu_sc as plsc

N_NODES = 10000
D_IN = 128
HID = 128
HEADS = 8
D_HEAD = HID // HEADS
E_EDGES = 160000

NC = 2
NS = 16
NW = NC * NS

CHUNK = 32
NCHUNKS = E_EDGES // CHUNK
ZROWS = 200
NZCHUNKS = N_NODES // ZROWS
ZB = 8
TPACK = N_NODES // 8
TPAD = 1280
TSTAGE = TPAD // NS
EBCH = 16
EBLK = EBCH * CHUNK
EPAD = 157 * CHUNK * NW - E_EDGES + EBLK

_f32 = jnp.float32


_BLK1 = 1000
_G1 = N_NODES // _BLK1


def _proj_body(xa, xp, Wa, ba, Wp, bp, asw, adw, asr, adr, asc, adc,
               ha_o, hp_o, tw, tr, tc):
    ha = jnp.dot(xa[...], Wa[...], preferred_element_type=_f32) + ba[...]
    hp = jnp.dot(xp[...], Wp[...], preferred_element_type=_f32) + bp[...]
    ha_o[...] = ha
    hp_o[...] = hp
    ii = lax.broadcasted_iota(jnp.int32, (HID, 16), 0) // D_HEAD
    jj = lax.broadcasted_iota(jnp.int32, (HID, 16), 1)
    S_lo = (ii == jj).astype(_f32)
    S_hi = (ii + HEADS == jj).astype(_f32)
    for out_ref, hs, a_s, hd, a_d in ((tw, ha, asw, hp, adw),
                                      (tr, hp, asr, ha, adr),
                                      (tc, hp, asc, hp, adc)):
        out_ref[...] = (jnp.dot(hs * a_s[...], S_lo, preferred_element_type=_f32)
                        + jnp.dot(hd * a_d[...], S_hi, preferred_element_type=_f32))


def _run_proj(xa, xp, Wa, ba, Wp, bp, avecs):
    row_spec = pl.BlockSpec((_BLK1, D_IN), lambda i: (i, 0))
    mat_spec = pl.BlockSpec((D_IN, HID), lambda i: (0, 0))
    vec_spec = pl.BlockSpec((1, HID), lambda i: (0, 0))
    tab_spec = pl.BlockSpec((_BLK1, 16), lambda i: (i, 0))
    return pl.pallas_call(
        _proj_body,
        grid=(_G1,),
        in_specs=[row_spec, row_spec, mat_spec, vec_spec, mat_spec, vec_spec]
                 + [vec_spec] * 6,
        out_specs=[row_spec, row_spec] + [tab_spec] * 3,
        out_shape=[jax.ShapeDtypeStruct((N_NODES, HID), _f32)] * 2
                  + [jax.ShapeDtypeStruct((N_NODES, 16), _f32)] * 3,
    )(xa, xp, Wa, ba.reshape(1, HID), Wp, bp.reshape(1, HID),
      *[a.reshape(1, HID) for a in avecs])


def _sc_edge_body(ha, hp, tw, tr, tc,
                  esw, edw, esr, edr, esc, edc,
                  U_out, d_out,
                  U_sh, d_sh, tab_sh,
                  sidxA, didxA, spbA, dpbA, scbA, dcbA, hbufA, ebufA,
                  sidxB, didxB, spbB, dpbB, scbB, dcbB, hbufB, ebufB,
                  ssb, sdb, zbuf, sidx_all, didx_all,
                  semA1, semB1, sem2, sem3, semAU, semAD, semBU, semBD):
    cid = lax.axis_index("c")
    sid = lax.axis_index("s")
    wid = sid * NC + cid

    zeros16 = jnp.zeros((16,), _f32)

    def zero_zbufs():
        def zrow(i, _):
            for g in range(HID // 16):
                zbuf[i, 16 * g:16 * (g + 1)] = zeros16
            return 0
        lax.fori_loop(0, ZB, zrow, 0)

    zero_zbufs()

    types = ((esw, edw, ha, tw),
             (esr, edr, hp, tr),
             (esc, edc, hp, tc))

    nrows_trip = (NZCHUNKS - sid + NS - 1) // NS
    perm = (lax.iota(jnp.int32, 16) + 8) & 15

    for t, (e_src, e_dst, h_src, st_tab) in enumerate(types):
        def zero_body(r, _):
            rb = (sid + r * NS) * ZROWS
            for z in range(ZROWS // ZB):
                pltpu.sync_copy(zbuf, U_sh.at[pl.ds(rb + z * ZB, ZB)])
            return 0
        lax.fori_loop(0, nrows_trip, zero_body, 0)
        def zero_den(z, _):
            pltpu.sync_copy(zbuf, d_sh.at[pl.ds(sid * TSTAGE + z * ZB, ZB)])
            return 0
        lax.fori_loop(0, TSTAGE // ZB, zero_den, 0)
        pltpu.sync_copy(st_tab.at[pl.ds(sid * TSTAGE, TSTAGE)],
                        tab_sh.at[pl.ds(sid * TSTAGE, TSTAGE)])
        plsc.subcore_barrier()

        ntr = 156 + (wid < 8).astype(jnp.int32)
        start_w = wid * 156 + jnp.minimum(wid, 8)

        bufsets = (
            (sidxA, didxA, spbA, dpbA, scbA, dcbA, hbufA, ebufA, semA1,
             semAU, semAD),
            (sidxB, didxB, spbB, dpbB, scbB, dcbB, hbufB, ebufB, semB1,
             semBU, semBD),
        )

        def prefetch(P, k):
            (sidx, didx, spb, dpb, scb, dcb, hbuf, ebuf, s1, sU, sD) = P

            @pl.when(k >= 2)
            def _():
                pltpu.make_async_copy(hbuf, U_sh.at[didx], sU).wait()
                pltpu.make_async_copy(ebuf, d_sh.at[dpb], sD).wait()

            @pl.when(k % EBCH == 0)
            def _():
                eb = start_w * CHUNK + (k // EBCH) * EBLK
                pltpu.sync_copy(e_src.at[pl.ds(eb, EBLK)], sidx_all)
                pltpu.sync_copy(e_dst.at[pl.ds(eb, EBLK)], didx_all)
            off = (k % EBCH) * CHUNK

            def grp(j, _):
                sv = sidx_all[pl.ds(off + j * 16, 16)]
                dv = didx_all[pl.ds(off + j * 16, 16)]
                sidx[pl.ds(j * 16, 16)] = sv
                didx[pl.ds(j * 16, 16)] = dv
                spb[pl.ds(j * 16, 16)] = lax.shift_right_logical(sv, 3)
                dpb[pl.ds(j * 16, 16)] = lax.shift_right_logical(dv, 3)
                scb[pl.ds(j * 16, 16)] = lax.shift_left(sv & 7, 4)
                dcb[pl.ds(j * 16, 16)] = lax.shift_left(dv & 7, 4)
                return 0
            lax.fori_loop(0, CHUNK // 16, grp, 0)

            pltpu.async_copy(h_src.at[sidx], hbuf, s1)

        def process(P):
            (sidx, didx, spb, dpb, scb, dcb, hbuf, ebuf, s1, sU, sD) = P
            cp2 = pltpu.async_copy(tab_sh.at[spb], ssb, sem2)
            cp3 = pltpu.async_copy(tab_sh.at[dpb], sdb, sem3)
            cp2.wait()
            cp3.wait()
            pltpu.make_async_copy(h_src.at[sidx], hbuf, s1).wait()

            def grp2(j, _):
                scv = scb[pl.ds(j * 16, 16)]
                dcv = dcb[pl.ds(j * 16, 16)]
                for jj in range(16):
                    i = j * 16 + jj
                    s_vec = ssb[i, pl.ds(scv[jj], 16)]
                    d_vec = sdb[i, pl.ds(dcv[jj], 16)]
                    a = s_vec + d_vec.at[perm].get(mode="promise_in_bounds")
                    a = jnp.where(a >= 0, a, 0.2 * a)
                    e = jnp.exp(a)
                    for g in range(HID // 16):
                        ebuf[i, 16 * g:16 * (g + 1)] = zeros16
                    ebuf[i, pl.ds(dcv[jj], 16)] = e
                    for g in range(HID // 16):
                        hbuf[i, 16 * g:16 * (g + 1)] = (
                            hbuf[i, 16 * g:16 * (g + 1)] * e[g])
                return 0
            lax.fori_loop(0, CHUNK // 16, grp2, 0)

            pltpu.async_copy(hbuf, U_sh.at[didx], sU, add=True)
            pltpu.async_copy(ebuf, d_sh.at[dpb], sD, add=True)

        prefetch(bufsets[0], 0)
        npairs = (ntr + 1) // 2

        def pair_body(m, _):
            k1 = 2 * m + 1

            @pl.when(k1 < ntr)
            def _():
                prefetch(bufsets[1], k1)
            process(bufsets[0])

            @pl.when(k1 + 1 < ntr)
            def _():
                prefetch(bufsets[0], k1 + 1)

            @pl.when(k1 < ntr)
            def _():
                process(bufsets[1])
            return 0

        lax.fori_loop(0, npairs, pair_body, 0)

        for (sidx, didx, spb, dpb, scb, dcb, hbuf, ebuf, s1, sU, sD) in bufsets:
            pltpu.make_async_copy(hbuf, U_sh.at[didx], sU).wait()
            pltpu.make_async_copy(ebuf, d_sh.at[dpb], sD).wait()
        plsc.subcore_barrier()

        def wb_body(r, _):
            rb = (sid + r * NS) * ZROWS
            pltpu.sync_copy(U_sh.at[pl.ds(rb, ZROWS)], U_out.at[t, cid, pl.ds(rb, ZROWS)])
            return 0
        lax.fori_loop(0, nrows_trip, wb_body, 0)
        pltpu.sync_copy(d_sh.at[pl.ds(sid * TSTAGE, TSTAGE)],
                        d_out.at[t, cid, pl.ds(sid * TSTAGE, TSTAGE)])


def _run_sc_edges(ha, hp, tables, edges):
    mesh = plsc.VectorSubcoreMesh(core_axis_name="c", subcore_axis_name="s",
                                  num_cores=NC, num_subcores=NS)
    kern = pl.kernel(
        _sc_edge_body,
        out_type=(jax.ShapeDtypeStruct((3, NC, N_NODES, HID), _f32),
                  jax.ShapeDtypeStruct((3, NC, TPAD, HID), _f32)),
        mesh=mesh,
        scratch_types=(
            pltpu.VMEM_SHARED((N_NODES, HID), _f32),
            pltpu.VMEM_SHARED((TPAD, HID), _f32),
            pltpu.VMEM_SHARED((TPAD, HID), _f32),
        ) + tuple(
            [pltpu.VMEM((CHUNK,), jnp.int32)] * 6
            + [pltpu.VMEM((CHUNK, HID), _f32)] * 2
            + [pltpu.VMEM((CHUNK,), jnp.int32)] * 6
            + [pltpu.VMEM((CHUNK, HID), _f32)] * 2
            + [pltpu.VMEM((CHUNK, HID), _f32)] * 2
        ) + (
            pltpu.VMEM((ZB, HID), _f32),
            pltpu.VMEM((EBLK,), jnp.int32),
            pltpu.VMEM((EBLK,), jnp.int32),
            pltpu.SemaphoreType.DMA,
            pltpu.SemaphoreType.DMA,
            pltpu.SemaphoreType.DMA,
            pltpu.SemaphoreType.DMA,
            pltpu.SemaphoreType.DMA,
            pltpu.SemaphoreType.DMA,
            pltpu.SemaphoreType.DMA,
            pltpu.SemaphoreType.DMA,
        ),
    )
    return kern(ha, hp, *tables, *edges)


_BLK2 = 1000
_G2 = N_NODES // _BLK2


def _combine_body(U, den, kW, kb, oa, ow, oc, psw, psc):
    kk = lax.broadcasted_iota(jnp.int32, (16, HID), 0)
    jj = lax.broadcasted_iota(jnp.int32, (16, HID), 1) // D_HEAD
    St = (kk == jj).astype(_f32)
    outs = []
    for t in range(3):
        Ut = U[t, 0] + U[t, 1]
        dent = den[t, 0] + den[t, 1]
        r = 1.0 / (dent + 1e-16)
        rexp = jnp.dot(r, St, preferred_element_type=_f32)
        outs.append(jnp.maximum(Ut * rexp, 0.0))
    ow[...] = outs[0]
    oa[...] = outs[1]
    oc[...] = outs[2]
    gw = jnp.tanh(jnp.dot(outs[0], kW[...], preferred_element_type=_f32) + kb[...])
    gc = jnp.tanh(jnp.dot(outs[2], kW[...], preferred_element_type=_f32) + kb[...])
    psw[...] = jnp.sum(gw, axis=0, keepdims=True)[None]
    psc[...] = jnp.sum(gc, axis=0, keepdims=True)[None]


def _run_combine(U, den, kW, kb):
    U_spec = pl.BlockSpec((3, NC, _BLK2, HID), lambda i: (0, 0, i, 0))
    d_spec = pl.BlockSpec((3, NC, _BLK2, 16), lambda i: (0, 0, i, 0))
    mat_spec = pl.BlockSpec((HID, HID), lambda i: (0, 0))
    vec_spec = pl.BlockSpec((1, HID), lambda i: (0, 0))
    row_spec = pl.BlockSpec((_BLK2, HID), lambda i: (i, 0))
    ps_spec = pl.BlockSpec((1, 1, HID), lambda i: (i, 0, 0))
    return pl.pallas_call(
        _combine_body,
        grid=(_G2,),
        in_specs=[U_spec, d_spec, mat_spec, vec_spec],
        out_specs=[row_spec, row_spec, row_spec, ps_spec, ps_spec],
        out_shape=[jax.ShapeDtypeStruct((N_NODES, HID), _f32)] * 3
                  + [jax.ShapeDtypeStruct((_G2, 1, HID), _f32)] * 2,
    )(U, den, kW, kb.reshape(1, HID))


def _final_body(ow, oc, psw, psc, q, op):
    qv = q[...]
    sw = jnp.sum(psw[...] * qv) / N_NODES
    sc = jnp.sum(psc[...] * qv) / N_NODES
    m = jnp.maximum(sw, sc)
    ew = jnp.exp(sw - m)
    ec = jnp.exp(sc - m)
    inv = 1.0 / (ew + ec)
    op[...] = (ew * inv) * ow[...] + (ec * inv) * oc[...]


def _run_final(ow, oc, psw, psc, q):
    row_spec = pl.BlockSpec((_BLK2, HID), lambda i: (i, 0))
    ps_spec = pl.BlockSpec((_G2, 1, HID), lambda i: (0, 0, 0))
    vec_spec = pl.BlockSpec((1, HID), lambda i: (0, 0))
    return pl.pallas_call(
        _final_body,
        grid=(_G2,),
        in_specs=[row_spec, row_spec, ps_spec, ps_spec, vec_spec],
        out_specs=row_spec,
        out_shape=jax.ShapeDtypeStruct((N_NODES, HID), _f32),
    )(ow, oc, psw, psc, q.reshape(1, HID))


def kernel(x_author, x_paper, edge_index_writes, edge_index_rev_writes,
           edge_index_cites, W_author, b_author, W_paper, b_paper,
           a_src_writes, a_dst_writes, a_src_rev, a_dst_rev, a_src_cites,
           a_dst_cites, k_W, k_b, q):
    avecs = (a_src_writes, a_dst_writes, a_src_rev, a_dst_rev,
             a_src_cites, a_dst_cites)
    ha, hp, tw, tr, tc = _run_proj(
        x_author, x_paper, W_author, b_author, W_paper, b_paper, avecs)

    epad = lambda e: jnp.pad(e, (0, EPAD))
    edges = (epad(edge_index_writes[0]), epad(edge_index_writes[1]),
             epad(edge_index_rev_writes[0]), epad(edge_index_rev_writes[1]),
             epad(edge_index_cites[0]), epad(edge_index_cites[1]))
    pack = lambda tab: jnp.pad(tab.reshape(TPACK, 8 * 16), ((0, TPAD - TPACK), (0, 0)))
    U, den_packed = _run_sc_edges(ha, hp, (pack(tw), pack(tr), pack(tc)), edges)
    den = den_packed[:, :, :TPACK].reshape(3, NC, N_NODES, 16)

    oa, ow, oc, psw, psc = _run_combine(U, den, k_W, k_b)
    op = _run_final(ow, oc, psw, psc, q)
    return (oa, op)
